# baseline (speedup 1.0000x reference)
import functools

import jax
import jax.numpy as jnp
from jax import lax
from jax.experimental import pallas as pl
from jax.experimental.pallas import tpu as pltpu
from jax.experimental.pallas import tpu_sc as plsc

NC = 2
NS = 16
NW = NC * NS
CH = 128
LANES = 16


def _make_deg_kernel(n_pad, cpt):
    mesh = plsc.VectorSubcoreMesh(core_axis_name="c", subcore_axis_name="s")

    @functools.partial(
        pl.kernel,
        out_type=jax.ShapeDtypeStruct((NW, n_pad), jnp.float32),
        mesh=mesh,
        compiler_params=pltpu.CompilerParams(needs_layout_passes=False),
        scratch_types=[
            pltpu.VMEM((cpt, CH), jnp.int32),
            pltpu.VMEM((n_pad,), jnp.float32),
        ],
    )
    def deg_kernel(cat, out, dst_v, hist):
        c = lax.axis_index("c")
        s = lax.axis_index("s")
        wid = s * NC + c

        pltpu.sync_copy(cat.at[1, pl.ds(wid * cpt, cpt)], dst_v)

        def zero_body(i, carry):
            for u in range(4):
                hist[pl.ds(i * 4 * LANES + u * LANES, LANES)] = jnp.zeros(
                    (LANES,), jnp.float32)
            return carry

        lax.fori_loop(0, n_pad // (4 * LANES), zero_body, 0)

        ones = jnp.full((LANES,), 1.0, jnp.float32)

        def add_body(j, carry):
            for k in range(CH // LANES):
                idx = dst_v[j, pl.ds(k * LANES, LANES)]
                plsc.addupdate_scatter(hist, [idx], ones)
            return carry

        lax.fori_loop(0, cpt, add_body, 0)
        pltpu.sync_copy(hist, out.at[wid])

    return deg_kernel


def _make_edge_scatter_kernel(n_pad, cpt, d):
    mesh = plsc.VectorSubcoreMesh(core_axis_name="c", subcore_axis_name="s")
    rows_per_tile = n_pad // NS
    scc = cpt
    while NS * (2 * scc * CH + 2 * CH * d) + n_pad * d > 2_000_000:
        scc //= 2
    n_stages = cpt // scc

    @functools.partial(
        pl.kernel,
        out_type=jax.ShapeDtypeStruct((NC, n_pad, d), jnp.float32),
        mesh=mesh,
        compiler_params=pltpu.CompilerParams(needs_layout_passes=False),
        scratch_types=[
            pltpu.VMEM((scc, CH), jnp.int32),
            pltpu.VMEM((scc, CH), jnp.int32),
            pltpu.VMEM((CH, d), jnp.float32),
            pltpu.VMEM((CH, d), jnp.float32),
            pltpu.VMEM_SHARED((n_pad, d), jnp.float32),
            pltpu.SemaphoreType.DMA,
            pltpu.SemaphoreType.DMA,
        ],
    )
    def edge_kernel(hs, cat, out, src_v, dst_v, rows_a, rows_b,
                    acc, sem_a, sem_b):
        c = lax.axis_index("c")
        s = lax.axis_index("s")
        wid = s * NC + c
        bufs = (rows_a, rows_b)
        sems = (sem_a, sem_b)

        with jax.named_scope("acc_zero"):
            def zero_body(i, carry):
                for l in range(d // LANES):
                    rows_a[i, pl.ds(l * LANES, LANES)] = jnp.zeros(
                        (LANES,), jnp.float32)
                return carry

            lax.fori_loop(0, CH, zero_body, 0)

            base = s * rows_per_tile
            off = 0
            while off < rows_per_tile:
                sz = min(CH, rows_per_tile - off)
                pltpu.sync_copy(rows_a.at[pl.ds(0, sz)],
                                acc.at[pl.ds(base + off, sz)])
                off += sz
            plsc.subcore_barrier()

        with jax.named_scope("edge_loop"):
            for stage in range(n_stages):
                ebase = wid * cpt + stage * scc
                pltpu.sync_copy(cat.at[0, pl.ds(ebase, scc)], src_v)
                pltpu.sync_copy(cat.at[1, pl.ds(ebase, scc)], dst_v)

                for b in range(2):
                    pltpu.async_copy(hs.at[src_v.at[b]], bufs[b], sems[b])

                def pair_body(i, carry):
                    j = i * 2
                    for b in range(2):
                        jj = j + b
                        pltpu.make_async_copy(hs.at[src_v.at[jj]], bufs[b],
                                              sems[b]).wait()
                        pltpu.sync_copy(bufs[b], acc.at[dst_v.at[jj]],
                                        add=True)

                        @pl.when(jj + 2 < scc)
                        def _():
                            pltpu.async_copy(hs.at[src_v.at[jj + 2]],
                                             bufs[b], sems[b])
                    return carry

                lax.fori_loop(0, scc // 2, pair_body, 0)

        with jax.named_scope("acc_out"):
            plsc.subcore_barrier()
            pltpu.sync_copy(acc.at[pl.ds(base, rows_per_tile)],
                            out.at[c, pl.ds(base, rows_per_tile)])

    return edge_kernel


def _dinv_from_parts(degt_blk):
    deg = jnp.maximum(jnp.sum(degt_blk, axis=1), 1.0)
    return lax.rsqrt(deg)


def _tc_prescale_matmul(x, w, degt, blk):
    n, d = x.shape
    nw = degt.shape[1]

    def body(x_ref, w_ref, degt_ref, hs_ref):
        dinv = _dinv_from_parts(degt_ref[...])
        hs_ref[...] = jnp.dot(
            x_ref[...], w_ref[...],
            preferred_element_type=jnp.float32) * dinv[:, None]

    return pl.pallas_call(
        body,
        grid=(n // blk,),
        in_specs=[
            pl.BlockSpec((blk, d), lambda i: (i, 0)),
            pl.BlockSpec((d, d), lambda i: (0, 0)),
            pl.BlockSpec((blk, nw), lambda i: (i, 0)),
        ],
        out_specs=pl.BlockSpec((blk, d), lambda i: (i, 0)),
        out_shape=jax.ShapeDtypeStruct((n, d), jnp.float32),
    )(x, w, degt)


def _tc_mid_layer(parts, degt, b, w, blk, n):
    _, _, d = parts.shape
    nw = degt.shape[1]

    def body(p_ref, degt_ref, b_ref, w_ref, hs_ref):
        dinv = _dinv_from_parts(degt_ref[...])
        agg = p_ref[0] + p_ref[1]
        h1 = jnp.maximum(agg * dinv[:, None] + b_ref[...], 0.0)
        hs_ref[...] = jnp.dot(
            h1, w_ref[...], preferred_element_type=jnp.float32) * dinv[:, None]

    return pl.pallas_call(
        body,
        grid=(n // blk,),
        in_specs=[
            pl.BlockSpec((2, blk, d), lambda i: (0, i, 0)),
            pl.BlockSpec((blk, nw), lambda i: (i, 0)),
            pl.BlockSpec((1, d), lambda i: (0, 0)),
            pl.BlockSpec((d, d), lambda i: (0, 0)),
        ],
        out_specs=pl.BlockSpec((blk, d), lambda i: (i, 0)),
        out_shape=jax.ShapeDtypeStruct((n, d), jnp.float32),
    )(parts, degt, b, w)


def _tc_final(parts, degt, b, blk, n):
    _, _, d = parts.shape
    nw = degt.shape[1]

    def body(p_ref, degt_ref, b_ref, out_ref):
        dinv = _dinv_from_parts(degt_ref[...])
        agg = p_ref[0] + p_ref[1]
        out_ref[...] = agg * dinv[:, None] + b_ref[...]

    return pl.pallas_call(
        body,
        grid=(n // blk,),
        in_specs=[
            pl.BlockSpec((2, blk, d), lambda i: (0, i, 0)),
            pl.BlockSpec((blk, nw), lambda i: (i, 0)),
            pl.BlockSpec((1, d), lambda i: (0, 0)),
        ],
        out_specs=pl.BlockSpec((blk, d), lambda i: (i, 0)),
        out_shape=jax.ShapeDtypeStruct((n, d), jnp.float32),
    )(parts, degt, b)


def kernel(x, edge_index, W1, b1, W2, b2):
    info = plsc.get_sparse_core_info()
    assert (info.num_cores, info.num_subcores) == (NC, NS), (
        "kernel is laid out for 2 SparseCores x 16 subcores")
    n, d = x.shape
    e = edge_index.shape[1]

    n_pad = ((n + 16) + 127) // 128 * 128
    cpt = -(-e // (NW * CH * 8)) * 8
    e_pad = NW * cpt * CH
    pad = e_pad - e
    ei = edge_index.astype(jnp.int32)
    if pad:
        pad_src = ((jnp.arange(pad, dtype=jnp.int32) * 997) % n)
        pad_dst = n + jnp.arange(pad, dtype=jnp.int32) % (n_pad - n)
        if e % CH == 0:
            pad3 = jnp.stack(
                [pad_src.reshape(pad // CH, CH),
                 pad_dst.reshape(pad // CH, CH)])
            cat = jnp.concatenate(
                [ei.reshape(2, e // CH, CH), pad3], axis=1)
        else:
            cat = jnp.concatenate(
                [ei, jnp.stack([pad_src, pad_dst])],
                axis=1).reshape(2, e_pad // CH, CH)
    else:
        cat = ei.reshape(2, e // CH, CH)

    deg_p = _make_deg_kernel(n_pad, cpt)(cat)
    degt = deg_p.T

    blk = 1000 if n % 1000 == 0 else n
    b1r = b1.reshape(1, d)
    b2r = b2.reshape(1, d)

    edge_scatter = _make_edge_scatter_kernel(n_pad, cpt, d)

    hs1 = _tc_prescale_matmul(x, W1, degt, blk)
    p1 = edge_scatter(hs1, cat)
    hs2 = _tc_mid_layer(p1, degt, b1r, W2, blk, n)
    p2 = edge_scatter(hs2, cat)
    out = _tc_final(p2, degt, b2r, blk, n)
    return out


# speedup vs baseline: 1.0237x; 1.0237x over previous
import functools

import jax
import jax.numpy as jnp
from jax import lax
from jax.experimental import pallas as pl
from jax.experimental.pallas import tpu as pltpu
from jax.experimental.pallas import tpu_sc as plsc

NC = 2
NS = 16
NW = NC * NS
CH = 128
LANES = 16


def _make_deg_kernel(n_pad, cpt):
    mesh = plsc.VectorSubcoreMesh(core_axis_name="c", subcore_axis_name="s")

    @functools.partial(
        pl.kernel,
        out_type=jax.ShapeDtypeStruct((NW, n_pad), jnp.float32),
        mesh=mesh,
        compiler_params=pltpu.CompilerParams(needs_layout_passes=False),
        scratch_types=[
            pltpu.VMEM((cpt, CH), jnp.int32),
            pltpu.VMEM((n_pad,), jnp.float32),
        ],
    )
    def deg_kernel(cat, out, dst_v, hist):
        c = lax.axis_index("c")
        s = lax.axis_index("s")
        wid = s * NC + c

        pltpu.sync_copy(cat.at[1, pl.ds(wid * cpt, cpt)], dst_v)

        def zero_body(i, carry):
            for u in range(4):
                hist[pl.ds(i * 4 * LANES + u * LANES, LANES)] = jnp.zeros(
                    (LANES,), jnp.float32)
            return carry

        lax.fori_loop(0, n_pad // (4 * LANES), zero_body, 0)

        ones = jnp.full((LANES,), 1.0, jnp.float32)

        def add_body(j, carry):
            for k in range(CH // LANES):
                idx = dst_v[j, pl.ds(k * LANES, LANES)]
                plsc.addupdate_scatter(hist, [idx], ones)
            return carry

        lax.fori_loop(0, cpt, add_body, 0)
        pltpu.sync_copy(hist, out.at[wid])

    return deg_kernel


def _make_edge_scatter_kernel(n_pad, cpt, d):
    mesh = plsc.VectorSubcoreMesh(core_axis_name="c", subcore_axis_name="s")
    rows_per_tile = n_pad // NS
    scc = cpt
    while NS * (2 * scc * CH + 2 * CH * d) + n_pad * d > 2_000_000:
        scc //= 2
    n_stages = cpt // scc

    @functools.partial(
        pl.kernel,
        out_type=jax.ShapeDtypeStruct((NC, n_pad, d), jnp.float32),
        mesh=mesh,
        compiler_params=pltpu.CompilerParams(needs_layout_passes=False),
        scratch_types=[
            pltpu.VMEM((scc, CH), jnp.int32),
            pltpu.VMEM((scc, CH), jnp.int32),
            pltpu.VMEM((CH, d), jnp.float32),
            pltpu.VMEM((CH, d), jnp.float32),
            pltpu.VMEM_SHARED((n_pad, d), jnp.float32),
            pltpu.SemaphoreType.DMA,
            pltpu.SemaphoreType.DMA,
        ],
    )
    def edge_kernel(hs, cat, out, src_v, dst_v, rows_a, rows_b,
                    acc, sem_a, sem_b):
        c = lax.axis_index("c")
        s = lax.axis_index("s")
        wid = s * NC + c
        bufs = (rows_a, rows_b)
        sems = (sem_a, sem_b)

        with jax.named_scope("acc_zero"):
            def zero_body(i, carry):
                for l in range(d // LANES):
                    rows_a[i, pl.ds(l * LANES, LANES)] = jnp.zeros(
                        (LANES,), jnp.float32)
                return carry

            lax.fori_loop(0, CH, zero_body, 0)

            base = s * rows_per_tile
            off = 0
            while off < rows_per_tile:
                sz = min(CH, rows_per_tile - off)
                pltpu.sync_copy(rows_a.at[pl.ds(0, sz)],
                                acc.at[pl.ds(base + off, sz)])
                off += sz
            plsc.subcore_barrier()

        with jax.named_scope("edge_loop"):
            for stage in range(n_stages):
                ebase = wid * cpt + stage * scc
                pltpu.sync_copy(cat.at[0, pl.ds(ebase, scc)], src_v)
                pltpu.sync_copy(cat.at[1, pl.ds(ebase, scc)], dst_v)

                for b in range(2):
                    pltpu.async_copy(hs.at[src_v.at[b]], bufs[b], sems[b])

                def pair_body(i, carry):
                    j = i * 2
                    for b in range(2):
                        jj = j + b
                        pltpu.make_async_copy(hs.at[src_v.at[jj]], bufs[b],
                                              sems[b]).wait()
                        pltpu.sync_copy(bufs[b], acc.at[dst_v.at[jj]],
                                        add=True)

                        @pl.when(jj + 2 < scc)
                        def _():
                            pltpu.async_copy(hs.at[src_v.at[jj + 2]],
                                             bufs[b], sems[b])
                    return carry

                lax.fori_loop(0, scc // 2, pair_body, 0)

        with jax.named_scope("acc_out"):
            plsc.subcore_barrier()
            pltpu.sync_copy(acc.at[pl.ds(base, rows_per_tile)],
                            out.at[c, pl.ds(base, rows_per_tile)])

    return edge_kernel


def _dinv_from_parts(degt_blk):
    deg = jnp.maximum(jnp.sum(degt_blk, axis=1), 1.0)
    return lax.rsqrt(deg)


def _tc_prescale_matmul(x, w, degt, blk):
    n, d = x.shape
    nw = degt.shape[1]

    def body(x_ref, w_ref, degt_ref, hs_ref):
        dinv = _dinv_from_parts(degt_ref[...])
        hs_ref[...] = jnp.dot(
            x_ref[...], w_ref[...],
            preferred_element_type=jnp.float32) * dinv[:, None]

    return pl.pallas_call(
        body,
        grid=(n // blk,),
        in_specs=[
            pl.BlockSpec((blk, d), lambda i: (i, 0)),
            pl.BlockSpec((d, d), lambda i: (0, 0)),
            pl.BlockSpec((blk, nw), lambda i: (i, 0)),
        ],
        out_specs=pl.BlockSpec((blk, d), lambda i: (i, 0)),
        out_shape=jax.ShapeDtypeStruct((n, d), jnp.float32),
    )(x, w, degt)


def _tc_mid_layer(parts, degt, b, w, blk, n):
    _, _, d = parts.shape
    nw = degt.shape[1]

    def body(p_ref, degt_ref, b_ref, w_ref, hs_ref):
        dinv = _dinv_from_parts(degt_ref[...])
        agg = p_ref[0] + p_ref[1]
        h1 = jnp.maximum(agg * dinv[:, None] + b_ref[...], 0.0)
        hs_ref[...] = jnp.dot(
            h1, w_ref[...], preferred_element_type=jnp.float32) * dinv[:, None]

    return pl.pallas_call(
        body,
        grid=(n // blk,),
        in_specs=[
            pl.BlockSpec((2, blk, d), lambda i: (0, i, 0)),
            pl.BlockSpec((blk, nw), lambda i: (i, 0)),
            pl.BlockSpec((1, d), lambda i: (0, 0)),
            pl.BlockSpec((d, d), lambda i: (0, 0)),
        ],
        out_specs=pl.BlockSpec((blk, d), lambda i: (i, 0)),
        out_shape=jax.ShapeDtypeStruct((n, d), jnp.float32),
    )(parts, degt, b, w)


def _tc_final(parts, degt, b, blk, n):
    _, _, d = parts.shape
    nw = degt.shape[1]

    def body(p_ref, degt_ref, b_ref, out_ref):
        dinv = _dinv_from_parts(degt_ref[...])
        agg = p_ref[0] + p_ref[1]
        out_ref[...] = agg * dinv[:, None] + b_ref[...]

    return pl.pallas_call(
        body,
        grid=(n // blk,),
        in_specs=[
            pl.BlockSpec((2, blk, d), lambda i: (0, i, 0)),
            pl.BlockSpec((blk, nw), lambda i: (i, 0)),
            pl.BlockSpec((1, d), lambda i: (0, 0)),
        ],
        out_specs=pl.BlockSpec((blk, d), lambda i: (i, 0)),
        out_shape=jax.ShapeDtypeStruct((n, d), jnp.float32),
    )(parts, degt, b)


def kernel(x, edge_index, W1, b1, W2, b2):
    info = plsc.get_sparse_core_info()
    assert (info.num_cores, info.num_subcores) == (NC, NS), (
        "kernel is laid out for 2 SparseCores x 16 subcores")
    n, d = x.shape
    e = edge_index.shape[1]

    n_pad = ((n + 16) + 127) // 128 * 128
    cpt = -(-e // (NW * CH * 8)) * 8
    e_pad = NW * cpt * CH
    pad = e_pad - e
    ei = edge_index.astype(jnp.int32)
    if pad:
        pad_src = ((jnp.arange(pad, dtype=jnp.int32) * 997) % n)
        pad_dst = n + jnp.arange(pad, dtype=jnp.int32) % (n_pad - n)
        if e % CH == 0:
            pad3 = jnp.stack(
                [pad_src.reshape(pad // CH, CH),
                 pad_dst.reshape(pad // CH, CH)])
            cat = jnp.concatenate(
                [ei.reshape(2, e // CH, CH), pad3], axis=1)
        else:
            cat = jnp.concatenate(
                [ei, jnp.stack([pad_src, pad_dst])],
                axis=1).reshape(2, e_pad // CH, CH)
    else:
        cat = ei.reshape(2, e // CH, CH)

    deg_p = _make_deg_kernel(n_pad, cpt)(cat)
    degt = deg_p.T

    blk = 2000 if n % 2000 == 0 else n
    b1r = b1.reshape(1, d)
    b2r = b2.reshape(1, d)

    edge_scatter = _make_edge_scatter_kernel(n_pad, cpt, d)

    hs1 = _tc_prescale_matmul(x, W1, degt, blk)
    p1 = edge_scatter(hs1, cat)
    hs2 = _tc_mid_layer(p1, degt, b1r, W2, blk, n)
    p2 = edge_scatter(hs2, cat)
    out = _tc_final(p2, degt, b2r, blk, n)
    return out
